# baseline (device time: 29828 ns/iter reference)
def kernel(x, A, B, C):
    import jax
    import jax.numpy as jnp
    from jax import lax
    from jax.experimental import pallas as pl
    from jax.experimental.pallas import tpu as pltpu

    Bdim, S, D = x.shape
    N = A.shape[1]
    TB = 8
    Dh = D // 2

    A_t = A.T

    def body(x_ref, a_ref, b_ref, c_ref, out_ref, hp_ref,
             hand_send_sem, hand_recv_sem, ex_send_sem, ex_recv_sem):
        my_x = lax.axis_index("x")
        my_y = lax.axis_index("y")
        d0 = pl.multiple_of(my_y * Dh, Dh)

        dA = jnp.exp(a_ref[:, pl.ds(d0, Dh)])[None]

        @pl.when(my_x == 1)
        def _():
            recv = pltpu.make_async_remote_copy(
                src_ref=hp_ref,
                dst_ref=hp_ref,
                send_sem=hand_send_sem,
                recv_sem=hand_recv_sem,
                device_id=(0, my_y),
                device_id_type=pl.DeviceIdType.MESH,
            )
            recv.wait_recv()

        zero = jnp.zeros((Bdim, N, Dh), jnp.float32)
        h0 = jnp.where(my_x == 0, zero, hp_ref[...])

        nblk = S // TB

        def blk(i, h):
            t0 = pl.multiple_of(i * TB, TB)
            xblk = x_ref[:, pl.ds(t0, TB), pl.ds(d0, Dh)]
            bblk = b_ref[:, pl.ds(t0, TB), :]
            cblk = c_ref[:, pl.ds(t0, TB), :]
            ys = []
            for j in range(TB):
                xt = xblk[:, j, :]
                bt = bblk[:, j, :]
                ct = cblk[:, j, :]
                h = h * dA + xt[:, None, :] * bt[:, :, None]
                ys.append(jnp.sum(h * ct[:, :, None], axis=1))
            out_ref[:, pl.ds(t0, TB), pl.ds(d0, Dh)] = jnp.stack(ys, axis=1)
            return h

        h_final = lax.fori_loop(0, nblk, blk, h0)

        @pl.when(my_x == 0)
        def _():
            hp_ref[...] = h_final
            send = pltpu.make_async_remote_copy(
                src_ref=hp_ref,
                dst_ref=hp_ref,
                send_sem=hand_send_sem,
                recv_sem=hand_recv_sem,
                device_id=(1, my_y),
                device_id_type=pl.DeviceIdType.MESH,
            )
            send.start()
            send.wait_send()

        ex = pltpu.make_async_remote_copy(
            src_ref=out_ref.at[:, :, pl.ds(d0, Dh)],
            dst_ref=out_ref.at[:, :, pl.ds(d0, Dh)],
            send_sem=ex_send_sem,
            recv_sem=ex_recv_sem,
            device_id=(my_x, 1 - my_y),
            device_id_type=pl.DeviceIdType.MESH,
        )
        ex.start()
        ex.wait()

    return pl.pallas_call(
        body,
        out_shape=jax.ShapeDtypeStruct((Bdim, S, D), jnp.float32),
        in_specs=[pl.BlockSpec(memory_space=pltpu.VMEM)] * 4,
        out_specs=pl.BlockSpec(memory_space=pltpu.VMEM),
        scratch_shapes=[
            pltpu.VMEM((Bdim, N, Dh), jnp.float32),
            pltpu.SemaphoreType.DMA,
            pltpu.SemaphoreType.DMA,
            pltpu.SemaphoreType.DMA,
            pltpu.SemaphoreType.DMA,
        ],
    )(x, A_t, B, C)


# device time: 24191 ns/iter; 1.2330x vs baseline; 1.2330x over previous
def kernel(x, A, B, C):
    import jax
    import jax.numpy as jnp
    from jax import lax
    from jax.experimental import pallas as pl
    from jax.experimental.pallas import tpu as pltpu

    Bdim, S, D = x.shape
    N = A.shape[1]
    TB = 8

    A_t = A.T

    def body(x_ref, a_ref, b_ref, c_ref, out_ref, hp_ref, send_sem, recv_sem):
        my_x = lax.axis_index("x")
        my_y = lax.axis_index("y")

        dA = jnp.exp(a_ref[...])[None]

        @pl.when(my_x == 1)
        def _():
            recv = pltpu.make_async_remote_copy(
                src_ref=hp_ref,
                dst_ref=hp_ref,
                send_sem=send_sem,
                recv_sem=recv_sem,
                device_id=(0, my_y),
                device_id_type=pl.DeviceIdType.MESH,
            )
            recv.wait_recv()

        zero = jnp.zeros((Bdim, N, D), jnp.float32)
        h0 = jnp.where(my_x == 0, zero, hp_ref[...])

        nblk = S // TB

        def blk(i, h):
            t0 = pl.multiple_of(i * TB, TB)
            xblk = x_ref[:, pl.ds(t0, TB), :]
            bblk = b_ref[:, pl.ds(t0, TB), :]
            cblk = c_ref[:, pl.ds(t0, TB), :]
            ublk = xblk[:, :, None, :] * bblk[:, :, :, None]
            hs = []
            for j in range(TB):
                h = h * dA + ublk[:, j]
                hs.append(h)
            hstk = jnp.stack(hs, axis=1)
            yblk = jnp.sum(hstk * cblk[:, :, :, None], axis=2)
            out_ref[:, pl.ds(t0, TB), :] = yblk
            return h

        h_final = lax.fori_loop(0, nblk, blk, h0)

        @pl.when(my_x == 0)
        def _():
            hp_ref[...] = h_final
            send = pltpu.make_async_remote_copy(
                src_ref=hp_ref,
                dst_ref=hp_ref,
                send_sem=send_sem,
                recv_sem=recv_sem,
                device_id=(1, my_y),
                device_id_type=pl.DeviceIdType.MESH,
            )
            send.start()
            send.wait_send()

    return pl.pallas_call(
        body,
        out_shape=jax.ShapeDtypeStruct((Bdim, S, D), jnp.float32),
        in_specs=[pl.BlockSpec(memory_space=pltpu.VMEM)] * 4,
        out_specs=pl.BlockSpec(memory_space=pltpu.VMEM),
        scratch_shapes=[
            pltpu.VMEM((Bdim, N, D), jnp.float32),
            pltpu.SemaphoreType.DMA,
            pltpu.SemaphoreType.DMA,
        ],
    )(x, A_t, B, C)


# device time: 18507 ns/iter; 1.6117x vs baseline; 1.3071x over previous
def kernel(x, A, B, C):
    import jax
    import jax.numpy as jnp
    from jax import lax
    from jax.experimental import pallas as pl
    from jax.experimental.pallas import tpu as pltpu

    Bdim, S, D = x.shape
    N = A.shape[1]
    TB = 8
    K = 4
    Lc = S // K

    A_t = A.T

    def body(x_ref, a_ref, b_ref, c_ref, out_ref, hp_ref, send_sem, recv_sem):
        my_x = lax.axis_index("x")
        my_y = lax.axis_index("y")

        a = a_ref[...]
        dA = jnp.exp(a)[None]
        zero = jnp.zeros((Bdim, N, D), jnp.float32)

        def blk(i, hs):
            t0 = pl.multiple_of(i * TB, TB)
            new_hs = []
            for k in range(K):
                tk = pl.multiple_of(k * Lc + i * TB, TB)
                xblk = x_ref[:, pl.ds(tk, TB), :]
                bblk = b_ref[:, pl.ds(tk, TB), :]
                cblk = c_ref[:, pl.ds(tk, TB), :]
                h = hs[k]
                ys = []
                for j in range(TB):
                    u = xblk[:, j, :][:, None, :] * bblk[:, j, :][:, :, None]
                    h = h * dA + u
                    ys.append(jnp.sum(h * cblk[:, j, :][:, :, None], axis=1))
                out_ref[:, pl.ds(tk, TB), :] = jnp.stack(ys, axis=1)
                new_hs.append(h)
            return tuple(new_hs)

        hf = lax.fori_loop(0, Lc // TB, blk, (zero,) * K)

        Q = jnp.exp(a * Lc)[None]
        h_in_local = [zero]
        for k in range(1, K):
            h_in_local.append(h_in_local[k - 1] * Q + hf[k - 1])
        my_final = h_in_local[K - 1] * Q + hf[K - 1]

        @pl.when(my_x == 0)
        def _():
            hp_ref[...] = my_final
            send = pltpu.make_async_remote_copy(
                src_ref=hp_ref,
                dst_ref=hp_ref,
                send_sem=send_sem,
                recv_sem=recv_sem,
                device_id=(1, my_y),
                device_id_type=pl.DeviceIdType.MESH,
            )
            send.start()
            send.wait_send()

        @pl.when(my_x == 1)
        def _():
            recv = pltpu.make_async_remote_copy(
                src_ref=hp_ref,
                dst_ref=hp_ref,
                send_sem=send_sem,
                recv_sem=recv_sem,
                device_id=(0, my_y),
                device_id_type=pl.DeviceIdType.MESH,
            )
            recv.wait_recv()

        hp = jnp.where(my_x == 0, zero, hp_ref[...])

        h_in = [
            h_in_local[k] + (jnp.exp(a * (k * Lc))[None] * hp if k else hp)
            for k in range(K)
        ]

        def corr(i, carry):
            t0 = pl.multiple_of(i * TB, TB)
            for k in range(K):
                tk = pl.multiple_of(k * Lc + i * TB, TB)
                cblk = c_ref[:, pl.ds(tk, TB), :]
                ys = []
                for j in range(TB):
                    e = jnp.exp(a * (t0 + j + 1).astype(jnp.float32))
                    g = h_in[k] * e[None]
                    ys.append(jnp.sum(g * cblk[:, j, :][:, :, None], axis=1))
                cur = out_ref[:, pl.ds(tk, TB), :]
                out_ref[:, pl.ds(tk, TB), :] = cur + jnp.stack(ys, axis=1)
            return carry

        lax.fori_loop(0, Lc // TB, corr, 0)

    return pl.pallas_call(
        body,
        out_shape=jax.ShapeDtypeStruct((Bdim, S, D), jnp.float32),
        in_specs=[pl.BlockSpec(memory_space=pltpu.VMEM)] * 4,
        out_specs=pl.BlockSpec(memory_space=pltpu.VMEM),
        scratch_shapes=[
            pltpu.VMEM((Bdim, N, D), jnp.float32),
            pltpu.SemaphoreType.DMA,
            pltpu.SemaphoreType.DMA,
        ],
    )(x, A_t, B, C)


# device time: 18495 ns/iter; 1.6128x vs baseline; 1.0006x over previous
def kernel(x, A, B, C):
    import jax
    import jax.numpy as jnp
    from jax import lax
    from jax.experimental import pallas as pl
    from jax.experimental.pallas import tpu as pltpu

    Bdim, S, D = x.shape
    N = A.shape[1]
    TB = 8
    K = 8
    Lc = S // K

    A_t = A.T

    def body(x_ref, a_ref, b_ref, c_ref, out_ref, hp_ref, send_sem, recv_sem):
        my_x = lax.axis_index("x")
        my_y = lax.axis_index("y")

        a = a_ref[...]
        dA = jnp.exp(a)[None]
        zero = jnp.zeros((Bdim, N, D), jnp.float32)

        def blk(i, hs):
            t0 = pl.multiple_of(i * TB, TB)
            new_hs = []
            for k in range(K):
                tk = pl.multiple_of(k * Lc + i * TB, TB)
                xblk = x_ref[:, pl.ds(tk, TB), :]
                bblk = b_ref[:, pl.ds(tk, TB), :]
                cblk = c_ref[:, pl.ds(tk, TB), :]
                h = hs[k]
                ys = []
                for j in range(TB):
                    u = xblk[:, j, :][:, None, :] * bblk[:, j, :][:, :, None]
                    h = h * dA + u
                    ys.append(jnp.sum(h * cblk[:, j, :][:, :, None], axis=1))
                out_ref[:, pl.ds(tk, TB), :] = jnp.stack(ys, axis=1)
                new_hs.append(h)
            return tuple(new_hs)

        hf = lax.fori_loop(0, Lc // TB, blk, (zero,) * K)

        Q = jnp.exp(a * Lc)[None]
        h_in_local = [zero]
        for k in range(1, K):
            h_in_local.append(h_in_local[k - 1] * Q + hf[k - 1])
        my_final = h_in_local[K - 1] * Q + hf[K - 1]

        @pl.when(my_x == 0)
        def _():
            hp_ref[...] = my_final
            send = pltpu.make_async_remote_copy(
                src_ref=hp_ref,
                dst_ref=hp_ref,
                send_sem=send_sem,
                recv_sem=recv_sem,
                device_id=(1, my_y),
                device_id_type=pl.DeviceIdType.MESH,
            )
            send.start()
            send.wait_send()

        @pl.when(my_x == 1)
        def _():
            recv = pltpu.make_async_remote_copy(
                src_ref=hp_ref,
                dst_ref=hp_ref,
                send_sem=send_sem,
                recv_sem=recv_sem,
                device_id=(0, my_y),
                device_id_type=pl.DeviceIdType.MESH,
            )
            recv.wait_recv()

        hp = jnp.where(my_x == 0, zero, hp_ref[...])

        h_in = [
            h_in_local[k] + (jnp.exp(a * (k * Lc))[None] * hp if k else hp)
            for k in range(K)
        ]

        def corr(i, carry):
            t0 = pl.multiple_of(i * TB, TB)
            es = [
                jnp.exp(a * (t0 + j + 1).astype(jnp.float32))[None]
                for j in range(TB)
            ]
            for k in range(K):
                tk = pl.multiple_of(k * Lc + i * TB, TB)
                cblk = c_ref[:, pl.ds(tk, TB), :]
                ys = []
                for j in range(TB):
                    g = h_in[k] * es[j]
                    ys.append(jnp.sum(g * cblk[:, j, :][:, :, None], axis=1))
                cur = out_ref[:, pl.ds(tk, TB), :]
                out_ref[:, pl.ds(tk, TB), :] = cur + jnp.stack(ys, axis=1)
            return carry

        lax.fori_loop(0, Lc // TB, corr, 0)

    return pl.pallas_call(
        body,
        out_shape=jax.ShapeDtypeStruct((Bdim, S, D), jnp.float32),
        in_specs=[pl.BlockSpec(memory_space=pltpu.VMEM)] * 4,
        out_specs=pl.BlockSpec(memory_space=pltpu.VMEM),
        scratch_shapes=[
            pltpu.VMEM((Bdim, N, D), jnp.float32),
            pltpu.SemaphoreType.DMA,
            pltpu.SemaphoreType.DMA,
        ],
    )(x, A_t, B, C)
